# Initial kernel scaffold; baseline (speedup 1.0000x reference)
import jax
import jax.numpy as jnp
from jax.experimental import pallas as pl


def kernel(features, target, centers):
    raise NotImplementedError("write your pallas kernel here")


# speedup vs baseline: 1.8879x; 1.8879x over previous
import functools

import jax
import jax.numpy as jnp
from jax import lax
from jax.experimental import pallas as pl
from jax.experimental.pallas import tpu as pltpu
from jax.experimental.pallas import tpu_sc as plsc

_BATCH = 1024
_FEAT = 64
_NUM_CLASSES = 100000
_LANES = 16

_NC = 2
_NS = 16
_NW = _NC * _NS
_BPW = _BATCH // _NW


@functools.partial(
    pl.kernel,
    mesh=plsc.VectorSubcoreMesh(core_axis_name="c", subcore_axis_name="s"),
    compiler_params=pltpu.CompilerParams(
        needs_layout_passes=False, use_tc_tiling_on_sc=False),
    out_type=jax.ShapeDtypeStruct((_NW, _LANES), jnp.float32),
    scratch_types=[
        pltpu.VMEM((_BPW,), jnp.int32),
        pltpu.VMEM((_BPW, _FEAT), jnp.float32),
        pltpu.VMEM((_BPW, _FEAT), jnp.float32),
        pltpu.VMEM((_BPW, _LANES + 1), jnp.float32),
        pltpu.VMEM((_LANES,), jnp.float32),
        pltpu.SemaphoreType.DMA,
        pltpu.SemaphoreType.DMA,
    ],
)
def _center_loss_partials(feat_hbm, tgt_hbm, cent_hbm, out_hbm,
                          idx_v, f_v, c_v, d_v, o_v, fsem, gsem):
    wid = lax.axis_index("s") * _NC + lax.axis_index("c")
    base = wid * _BPW
    pltpu.sync_copy(tgt_hbm.at[pl.ds(base, _BPW)], idx_v)
    fcp = pltpu.async_copy(feat_hbm.at[pl.ds(base, _BPW)], f_v, fsem)
    gcp = pltpu.async_copy(cent_hbm.at[idx_v], c_v, gsem)
    fcp.wait()
    gcp.wait()
    for i in range(_BPW):
        acc = jnp.zeros((_LANES,), jnp.float32)
        for ch in range(_FEAT // _LANES):
            df = (f_v[i, pl.ds(ch * _LANES, _LANES)]
                  - c_v[i, pl.ds(ch * _LANES, _LANES)])
            acc = acc + df * df
        d_v[i, pl.ds(0, _LANES)] = acc
    lanes = lax.iota(jnp.int32, _LANES)
    partial = jnp.zeros((_LANES,), jnp.float32)
    for g in range(_BPW // _LANES):
        rows = lanes + jnp.int32(g * _LANES)
        tot = jnp.zeros((_LANES,), jnp.float32)
        for j in range(_LANES):
            cols = jnp.full((_LANES,), j, jnp.int32)
            tot = tot + plsc.load_gather(d_v, [rows, cols])
        tot = jnp.minimum(jnp.maximum(tot, jnp.float32(1e-12)),
                          jnp.float32(1e12))
        partial = partial + tot
    o_v[...] = partial
    pltpu.sync_copy(o_v, out_hbm.at[wid])


def kernel(features, target, centers):
    partials = _center_loss_partials(features, target, centers)
    zero_term = jnp.float32((_NUM_CLASSES - 1) * 1e-12)
    return jnp.sum(partials) / jnp.float32(_BATCH) + zero_term


# speedup vs baseline: 2.6820x; 1.4206x over previous
import functools

import jax
import jax.numpy as jnp
from jax import lax
from jax.experimental import pallas as pl
from jax.experimental.pallas import tpu as pltpu
from jax.experimental.pallas import tpu_sc as plsc

_BATCH = 1024
_FEAT = 64
_NUM_CLASSES = 100000
_LANES = 16

_NC = 2
_NS = 16
_NW = _NC * _NS
_BPW = _BATCH // _NW


@functools.partial(
    pl.kernel,
    mesh=plsc.VectorSubcoreMesh(core_axis_name="c", subcore_axis_name="s"),
    compiler_params=pltpu.CompilerParams(needs_layout_passes=False),
    out_type=jax.ShapeDtypeStruct((_NW, _LANES), jnp.float32),
    scratch_types=[
        pltpu.VMEM((_BPW,), jnp.int32),
        pltpu.VMEM((_BPW, _FEAT), jnp.float32),
        pltpu.VMEM((_BPW, _FEAT), jnp.float32),
        pltpu.VMEM((_BPW, _LANES + 1), jnp.float32),
        pltpu.VMEM((_LANES,), jnp.float32),
        pltpu.SemaphoreType.DMA,
        pltpu.SemaphoreType.DMA,
        pltpu.SemaphoreType.DMA,
    ],
)
def _center_loss_partials(feat_hbm, tgt_hbm, cent_hbm, out_hbm,
                          idx_v, f_v, c_v, d_v, o_v, fsem, gsem, isem):
    wid = lax.axis_index("s") * _NC + lax.axis_index("c")
    base = wid * _BPW
    pltpu.sync_copy(tgt_hbm.at[pl.ds(base, _BPW)], idx_v)
    fcp = pltpu.async_copy(feat_hbm.at[pl.ds(base, _BPW)], f_v, fsem)
    copies = []
    for g in range(_BPW // _LANES):
        tv = idx_v[pl.ds(g * _LANES, _LANES)]
        for i in range(_LANES):
            r = tv[i]
            copies.append(
                pltpu.async_copy(cent_hbm.at[r], c_v.at[g * _LANES + i], gsem))
    fcp.wait()
    for cp in copies:
        cp.wait()
    for i in range(_BPW):
        acc = jnp.zeros((_LANES,), jnp.float32)
        for ch in range(_FEAT // _LANES):
            df = (f_v[i, pl.ds(ch * _LANES, _LANES)]
                  - c_v[i, pl.ds(ch * _LANES, _LANES)])
            acc = acc + df * df
        d_v[i, pl.ds(0, _LANES)] = acc
    lanes = lax.iota(jnp.int32, _LANES)
    partial = jnp.zeros((_LANES,), jnp.float32)
    for g in range(_BPW // _LANES):
        rows = lanes + jnp.int32(g * _LANES)
        tot = jnp.zeros((_LANES,), jnp.float32)
        for j in range(_LANES):
            cols = jnp.full((_LANES,), j, jnp.int32)
            tot = tot + plsc.load_gather(d_v, [rows, cols])
        tot = jnp.minimum(jnp.maximum(tot, jnp.float32(1e-12)),
                          jnp.float32(1e12))
        partial = partial + tot
    o_v[...] = partial
    pltpu.sync_copy(o_v, out_hbm.at[wid])


def kernel(features, target, centers):
    partials = _center_loss_partials(features, target, centers)
    zero_term = jnp.float32((_NUM_CLASSES - 1) * 1e-12)
    return jnp.sum(partials) / jnp.float32(_BATCH) + zero_term


# speedup vs baseline: 4.0154x; 1.4972x over previous
import functools

import jax
import jax.numpy as jnp
from jax import lax
from jax.experimental import pallas as pl
from jax.experimental.pallas import tpu as pltpu
from jax.experimental.pallas import tpu_sc as plsc

_BATCH = 1024
_FEAT = 64
_NUM_CLASSES = 100000
_LANES = 16

_NC = 2
_NS = 16
_NW = _NC * _NS
_BPW = _BATCH // _NW
_TW = 128
_GRP = 4


@functools.partial(
    pl.kernel,
    mesh=plsc.VectorSubcoreMesh(core_axis_name="c", subcore_axis_name="s"),
    compiler_params=pltpu.CompilerParams(needs_layout_passes=False),
    out_type=jax.ShapeDtypeStruct((_NW, _LANES), jnp.float32),
    scratch_types=[
        pltpu.VMEM((_BPW,), jnp.int32),
        pltpu.VMEM((_BPW, _FEAT), jnp.float32),
        pltpu.VMEM((2 * _GRP, _FEAT, _TW), jnp.float32),
        pltpu.VMEM((_LANES,), jnp.float32),
        pltpu.SemaphoreType.DMA,
        pltpu.SemaphoreType.DMA,
        pltpu.SemaphoreType.DMA,
    ],
)
def _center_loss_partials(feat_hbm, tgt_hbm, ct_hbm, out_hbm,
                          idx_v, f_v, slab_v, o_v, fsem, gsem_a, gsem_b):
    wid = lax.axis_index("s") * _NC + lax.axis_index("c")
    base = wid * _BPW
    pltpu.sync_copy(tgt_hbm.at[pl.ds(base, _BPW)], idx_v)
    fcp = pltpu.async_copy(feat_hbm.at[pl.ds(base, _BPW)], f_v, fsem)

    tgt_rows = [None] * _BPW
    for g in range(_BPW // _LANES):
        tv = idx_v[pl.ds(g * _LANES, _LANES)]
        for i in range(_LANES):
            tgt_rows[g * _LANES + i] = tv[i]

    sems = [gsem_a, gsem_b]
    nbatch = _BPW // _GRP

    def fire_batch(b):
        cps = []
        for k in range(_GRP):
            r = tgt_rows[b * _GRP + k]
            col0 = pl.multiple_of((r // _TW) * _TW, _TW)
            cps.append(
                pltpu.async_copy(ct_hbm.at[:, pl.ds(col0, _TW)],
                                 slab_v.at[(b % 2) * _GRP + k],
                                 sems[b % 2]))
        return cps

    lanes = lax.iota(jnp.int32, _LANES)
    fcp.wait()
    inflight = fire_batch(0)
    acc = jnp.zeros((_LANES,), jnp.float32)
    for b in range(nbatch):
        nxt = fire_batch(b + 1) if b + 1 < nbatch else []
        for cp in inflight:
            cp.wait()
        for k in range(_GRP):
            i = b * _GRP + k
            r = tgt_rows[i]
            cloc = jnp.full((_LANES,), r % _TW, jnp.int32)
            sbuf = slab_v.at[(b % 2) * _GRP + k]
            for ch in range(_FEAT // _LANES):
                dims = lanes + jnp.int32(ch * _LANES)
                cvals = plsc.load_gather(sbuf, [dims, cloc])
                df = f_v[i, pl.ds(ch * _LANES, _LANES)] - cvals
                acc = acc + df * df
        inflight = nxt
    o_v[...] = acc
    pltpu.sync_copy(o_v, out_hbm.at[wid])


def kernel(features, target, centers):
    partials = _center_loss_partials(features, target, centers.T)
    zero_term = jnp.float32((_NUM_CLASSES - 1) * 1e-12)
    return jnp.sum(partials) / jnp.float32(_BATCH) + zero_term


# speedup vs baseline: 4.0232x; 1.0019x over previous
import functools

import jax
import jax.numpy as jnp
from jax import lax
from jax.experimental import pallas as pl
from jax.experimental.pallas import tpu as pltpu
from jax.experimental.pallas import tpu_sc as plsc

_BATCH = 1024
_FEAT = 64
_NUM_CLASSES = 100000
_LANES = 16

_NC = 2
_NS = 16
_NW = _NC * _NS
_BPW = _BATCH // _NW
_TW = 128
_GRP = 4


@functools.partial(
    pl.kernel,
    mesh=plsc.VectorSubcoreMesh(core_axis_name="c", subcore_axis_name="s"),
    compiler_params=pltpu.CompilerParams(
        needs_layout_passes=False, skip_device_barrier=True),
    out_type=jax.ShapeDtypeStruct((_NW, _LANES), jnp.float32),
    scratch_types=[
        pltpu.VMEM((_BPW,), jnp.int32),
        pltpu.VMEM((_BPW, _FEAT), jnp.float32),
        pltpu.VMEM((2 * _GRP, _FEAT, _TW), jnp.float32),
        pltpu.VMEM((_LANES,), jnp.float32),
        pltpu.SemaphoreType.DMA,
        pltpu.SemaphoreType.DMA,
        pltpu.SemaphoreType.DMA,
    ],
)
def _center_loss_partials(feat_hbm, tgt_hbm, ct_hbm, out_hbm,
                          idx_v, f_v, slab_v, o_v, fsem, gsem_a, gsem_b):
    wid = lax.axis_index("s") * _NC + lax.axis_index("c")
    base = wid * _BPW
    pltpu.sync_copy(tgt_hbm.at[pl.ds(base, _BPW)], idx_v)
    fcp = pltpu.async_copy(feat_hbm.at[pl.ds(base, _BPW)], f_v, fsem)

    tgt_rows = [None] * _BPW
    for g in range(_BPW // _LANES):
        tv = idx_v[pl.ds(g * _LANES, _LANES)]
        for i in range(_LANES):
            tgt_rows[g * _LANES + i] = tv[i]

    sems = [gsem_a, gsem_b]
    nbatch = _BPW // _GRP

    def fire_batch(b):
        cps = []
        for k in range(_GRP):
            r = tgt_rows[b * _GRP + k]
            col0 = pl.multiple_of((r // _TW) * _TW, _TW)
            cps.append(
                pltpu.async_copy(ct_hbm.at[:, pl.ds(col0, _TW)],
                                 slab_v.at[(b % 2) * _GRP + k],
                                 sems[b % 2]))
        return cps

    lanes = lax.iota(jnp.int32, _LANES)
    fcp.wait()
    inflight = fire_batch(0)
    acc = jnp.zeros((_LANES,), jnp.float32)
    for b in range(nbatch):
        nxt = fire_batch(b + 1) if b + 1 < nbatch else []
        for cp in inflight:
            cp.wait()
        for k in range(_GRP):
            i = b * _GRP + k
            r = tgt_rows[i]
            cloc = jnp.full((_LANES,), r % _TW, jnp.int32)
            sbuf = slab_v.at[(b % 2) * _GRP + k]
            for ch in range(_FEAT // _LANES):
                dims = lanes + jnp.int32(ch * _LANES)
                cvals = plsc.load_gather(sbuf, [dims, cloc])
                df = f_v[i, pl.ds(ch * _LANES, _LANES)] - cvals
                acc = acc + df * df
        inflight = nxt
    o_v[...] = acc
    pltpu.sync_copy(o_v, out_hbm.at[wid])


def kernel(features, target, centers):
    partials = _center_loss_partials(features, target, centers.T)
    zero_term = jnp.float32((_NUM_CLASSES - 1) * 1e-12)
    return jnp.sum(partials) / jnp.float32(_BATCH) + zero_term
